# baseline (speedup 1.0000x reference)
import functools

import jax
import jax.numpy as jnp
from jax import lax
from jax.experimental import pallas as pl
from jax.experimental.pallas import tpu as pltpu
from jax.experimental.pallas import tpu_sc as plsc

D = 64
L = 16
NC, NS = 2, 16
NW = NC * NS
SCALE = 8.0

B = 4096 * 200
BPW = B // NW
CHUNK = 1024
NSTEP = BPW // CHUNK

_mesh = plsc.VectorSubcoreMesh(core_axis_name="c", subcore_axis_name="s")


@functools.partial(
    pl.kernel,
    out_type=jax.ShapeDtypeStruct((B, D), jnp.float32),
    mesh=_mesh,
    scratch_types=[
        pltpu.VMEM((CHUNK,), jnp.int32),
        pltpu.VMEM((CHUNK, D), jnp.float32),
        pltpu.SemaphoreType.DMA,
    ],
    compiler_params=pltpu.CompilerParams(use_tc_tiling_on_sc=False),
)
def _emb_lookup(idx_hbm, table_hbm, out_hbm, idx_v, rows_v, sem):
    wid = lax.axis_index("s") * NC + lax.axis_index("c")
    base = wid * BPW

    def step(g, carry):
        off = base + g * CHUNK
        pltpu.sync_copy(idx_hbm.at[pl.ds(off, CHUNK)], idx_v)
        pltpu.async_copy(table_hbm.at[idx_v], rows_v, sem).wait()

        def scale_row(r, c2):
            for c in range(D // L):
                sl = pl.ds(c * L, L)
                rows_v[r, sl] = rows_v[r, sl] * SCALE
            return c2

        lax.fori_loop(0, CHUNK, scale_row, 0)
        pltpu.sync_copy(rows_v, out_hbm.at[pl.ds(off, CHUNK)])
        return carry

    lax.fori_loop(0, NSTEP, step, 0)


def kernel(x, weight):
    out = _emb_lookup(x.reshape(B), weight)
    return out.reshape(4096, 200, D)


# speedup vs baseline: 1.1085x; 1.1085x over previous
import functools

import jax
import jax.numpy as jnp
from jax import lax
from jax.experimental import pallas as pl
from jax.experimental.pallas import tpu as pltpu
from jax.experimental.pallas import tpu_sc as plsc

D = 64
L = 16
NC, NS = 2, 16
NW = NC * NS
SCALE = 8.0

B = 4096 * 200
BPW = B // NW
CHUNK = 800
NSTEP = BPW // CHUNK

_mesh = plsc.VectorSubcoreMesh(core_axis_name="c", subcore_axis_name="s")


@functools.partial(
    pl.kernel,
    out_type=jax.ShapeDtypeStruct((B, D), jnp.float32),
    mesh=_mesh,
    scratch_types=[
        pltpu.VMEM((CHUNK,), jnp.int32),
        pltpu.VMEM((CHUNK,), jnp.int32),
        pltpu.VMEM((CHUNK, D), jnp.float32),
        pltpu.VMEM((CHUNK, D), jnp.float32),
        pltpu.SemaphoreType.DMA,
        pltpu.SemaphoreType.DMA,
        pltpu.SemaphoreType.DMA,
        pltpu.SemaphoreType.DMA,
        pltpu.SemaphoreType.DMA,
        pltpu.SemaphoreType.DMA,
    ],
    compiler_params=pltpu.CompilerParams(use_tc_tiling_on_sc=False),
)
def _emb_lookup(idx_hbm, table_hbm, out_hbm,
                idx0, idx1, rows0, rows1,
                isem0, isem1, gsem0, gsem1, osem0, osem1):
    wid = lax.axis_index("s") * NC + lax.axis_index("c")
    base = wid * BPW
    idxb = (idx0, idx1)
    rowsb = (rows0, rows1)
    isem = (isem0, isem1)
    gsem = (gsem0, gsem1)
    osem = (osem0, osem1)

    def idx_start(g, j):
        pltpu.async_copy(idx_hbm.at[pl.ds(base + g * CHUNK, CHUNK)],
                         idxb[j], isem[j])

    def idx_wait(j):
        pltpu.make_async_copy(idx_hbm.at[pl.ds(base, CHUNK)],
                              idxb[j], isem[j]).wait()

    def gather_start(j):
        pltpu.async_copy(table_hbm.at[idxb[j]], rowsb[j], gsem[j])

    def gather_wait(j):
        pltpu.make_async_copy(table_hbm.at[idxb[j]], rowsb[j], gsem[j]).wait()

    def out_start(g, j):
        pltpu.async_copy(rowsb[j], out_hbm.at[pl.ds(base + g * CHUNK, CHUNK)],
                         osem[j])

    def out_wait(j):
        pltpu.make_async_copy(rowsb[j], out_hbm.at[pl.ds(base, CHUNK)],
                              osem[j]).wait()

    idx_start(0, 0)
    idx_start(1, 1)
    idx_wait(0)
    gather_start(0)

    def chunk_body(g, j):
        nj = 1 - j
        gather_wait(j)

        @pl.when(g + 2 < NSTEP)
        def _():
            idx_start(g + 2, j)

        @pl.when(g + 1 < NSTEP)
        def _():
            @pl.when(g >= 1)
            def _():
                out_wait(nj)
            idx_wait(nj)
            gather_start(nj)

        @plsc.parallel_loop(0, CHUNK, 1, unroll=4)
        def _(r):
            for c in range(D // L):
                sl = pl.ds(c * L, L)
                rowsb[j][r, sl] = rowsb[j][r, sl] * SCALE

        out_start(g, j)

    @pl.loop(0, NSTEP, step=2)
    def _(gg):
        chunk_body(gg, 0)
        chunk_body(gg + 1, 1)

    out_wait(0)
    out_wait(1)


def kernel(x, weight):
    out = _emb_lookup(x.reshape(B), weight)
    return out.reshape(4096, 200, D)


# speedup vs baseline: 1.2869x; 1.1610x over previous
import functools

import jax
import jax.numpy as jnp
from jax import lax
from jax.experimental import pallas as pl
from jax.experimental.pallas import tpu as pltpu
from jax.experimental.pallas import tpu_sc as plsc

D = 64
DP = 128
L = 16
NC, NS = 2, 16
NW = NC * NS
SCALE = 8.0

B = 4096 * 200
BPW = B // NW
CHUNK = 128
NSTEP = BPW // CHUNK

_mesh = plsc.VectorSubcoreMesh(core_axis_name="c", subcore_axis_name="s")


@functools.partial(
    pl.kernel,
    out_type=jax.ShapeDtypeStruct((B, D), jnp.float32),
    mesh=_mesh,
    scratch_types=[
        pltpu.VMEM((CHUNK,), jnp.int32),
        pltpu.VMEM((CHUNK,), jnp.int32),
        pltpu.VMEM((CHUNK, DP), jnp.float32),
        pltpu.VMEM((CHUNK, DP), jnp.float32),
        pltpu.VMEM((CHUNK, D), jnp.float32),
        pltpu.VMEM((CHUNK, D), jnp.float32),
        pltpu.SemaphoreType.DMA,
        pltpu.SemaphoreType.DMA,
        pltpu.SemaphoreType.DMA,
        pltpu.SemaphoreType.DMA,
        pltpu.SemaphoreType.DMA,
        pltpu.SemaphoreType.DMA,
    ],
)
def _emb_lookup(idx_hbm, table_hbm, out_hbm,
                idx0, idx1, pairs0, pairs1, outv0, outv1,
                isem0, isem1, gsem0, gsem1, osem0, osem1):
    wid = lax.axis_index("s") * NC + lax.axis_index("c")
    base = wid * BPW
    idxb = (idx0, idx1)
    rowsb = (pairs0, pairs1)
    outb = (outv0, outv1)
    isem = (isem0, isem1)
    gsem = (gsem0, gsem1)
    osem = (osem0, osem1)

    def idx_start(g, j):
        pltpu.async_copy(idx_hbm.at[pl.ds(base + g * CHUNK, CHUNK)],
                         idxb[j], isem[j])

    def idx_wait(j):
        pltpu.make_async_copy(idx_hbm.at[pl.ds(base, CHUNK)],
                              idxb[j], isem[j]).wait()

    def gather_start(j):
        pltpu.async_copy(table_hbm.at[idxb[j]], rowsb[j], gsem[j])

    def gather_wait(j):
        pltpu.make_async_copy(table_hbm.at[idxb[j]], rowsb[j], gsem[j]).wait()

    def out_start(g, j):
        pltpu.async_copy(outb[j], out_hbm.at[pl.ds(base + g * CHUNK, CHUNK)],
                         osem[j])

    def out_wait(j):
        pltpu.make_async_copy(outb[j], out_hbm.at[pl.ds(base, CHUNK)],
                              osem[j]).wait()

    idx_start(0, 0)
    idx_start(1, 1)
    idx_wait(0)
    gather_start(0)

    def chunk_body(g, j):
        nj = 1 - j
        gather_wait(j)

        @pl.when(g + 2 < NSTEP)
        def _():
            idx_start(g + 2, j)

        @pl.when(g + 1 < NSTEP)
        def _():
            idx_wait(nj)
            gather_start(nj)

        @pl.when(g >= 2)
        def _():
            out_wait(j)

        @plsc.parallel_loop(0, CHUNK, 1, unroll=4)
        def _(r):
            for c in range(D // L):
                sl = pl.ds(c * L, L)
                outb[j][r, sl] = rowsb[j][r, sl] * SCALE

        out_start(g, j)

    @pl.loop(0, NSTEP, step=2)
    def _(gg):
        chunk_body(gg, 0)
        chunk_body(gg + 1, 1)

    out_wait(0)
    out_wait(1)


def kernel(x, weight):
    wp = jnp.pad(weight, ((0, 0), (0, DP - D)))
    out = _emb_lookup(x.reshape(B), wp)
    return out.reshape(4096, 200, D)


# speedup vs baseline: 1.3552x; 1.0531x over previous
import functools

import jax
import jax.numpy as jnp
from jax import lax
from jax.experimental import pallas as pl
from jax.experimental.pallas import tpu as pltpu
from jax.experimental.pallas import tpu_sc as plsc

D = 64
DP = 128
L = 16
NC, NS = 2, 16
NW = NC * NS
SCALE = 8.0
NBUF = 4

B = 4096 * 200
BPW = B // NW
CHUNK = 128
NSTEP = BPW // CHUNK

_mesh = plsc.VectorSubcoreMesh(core_axis_name="c", subcore_axis_name="s")


@functools.partial(
    pl.kernel,
    out_type=jax.ShapeDtypeStruct((B, D), jnp.float32),
    mesh=_mesh,
    scratch_types=(
        [pltpu.VMEM((CHUNK,), jnp.int32) for _ in range(NBUF)]
        + [pltpu.VMEM((CHUNK, DP), jnp.float32) for _ in range(NBUF)]
        + [pltpu.VMEM((CHUNK, D), jnp.float32) for _ in range(2)]
        + [pltpu.SemaphoreType.DMA for _ in range(2 * NBUF + 2)]
    ),
    compiler_params=pltpu.CompilerParams(use_tc_tiling_on_sc=True),
)
def _emb_lookup(idx_hbm, table_hbm, out_hbm, *bufs):
    idxb = bufs[0:NBUF]
    rowsb = bufs[NBUF:2 * NBUF]
    outb = bufs[2 * NBUF:2 * NBUF + 2]
    isem = bufs[2 * NBUF + 2:3 * NBUF + 2]
    gsem = bufs[3 * NBUF + 2:4 * NBUF + 2]
    osem = bufs[4 * NBUF + 2:4 * NBUF + 4]

    wid = lax.axis_index("s") * NC + lax.axis_index("c")
    base = wid * BPW

    def idx_start(g, j):
        pltpu.async_copy(idx_hbm.at[pl.ds(base + g * CHUNK, CHUNK)],
                         idxb[j], isem[j])

    def idx_wait(j):
        pltpu.make_async_copy(idx_hbm.at[pl.ds(base, CHUNK)],
                              idxb[j], isem[j]).wait()

    def gather_start(j):
        pltpu.async_copy(table_hbm.at[idxb[j]], rowsb[j], gsem[j])

    def gather_wait(j):
        pltpu.make_async_copy(table_hbm.at[idxb[j]], rowsb[j], gsem[j]).wait()

    def out_start(g, jo):
        pltpu.async_copy(outb[jo],
                         out_hbm.at[pl.ds(base + g * CHUNK, CHUNK)], osem[jo])

    def out_wait(jo):
        pltpu.make_async_copy(outb[jo],
                              out_hbm.at[pl.ds(base, CHUNK)], osem[jo]).wait()

    for j in range(NBUF):
        idx_start(j, j)
    idx_wait(0)
    gather_start(0)
    idx_wait(1)
    gather_start(1)

    def chunk_body(g, j, jo):
        gather_wait(j)

        @pl.when(g + NBUF < NSTEP)
        def _():
            idx_start(g + NBUF, j)

        @pl.when(g + 2 < NSTEP)
        def _():
            j2 = (j + 2) % NBUF
            idx_wait(j2)
            gather_start(j2)

        @pl.when(g >= 2)
        def _():
            out_wait(jo)

        @plsc.parallel_loop(0, CHUNK, 1, unroll=4)
        def _(r):
            for c in range(D // L):
                sl = pl.ds(c * L, L)
                outb[jo][r, sl] = rowsb[j][r, sl] * SCALE

        out_start(g, jo)

    @pl.loop(0, NSTEP, step=NBUF)
    def _(gg):
        for j in range(NBUF):
            chunk_body(gg + j, j, j % 2)

    out_wait(0)
    out_wait(1)


def kernel(x, weight):
    wp = jnp.pad(weight, ((0, 0), (0, DP - D)))
    out = _emb_lookup(x.reshape(B), wp)
    return out.reshape(4096, 200, D)


# speedup vs baseline: 1.6623x; 1.2266x over previous
import functools

import jax
import jax.numpy as jnp
from jax import lax
from jax.experimental import pallas as pl
from jax.experimental.pallas import tpu as pltpu
from jax.experimental.pallas import tpu_sc as plsc

D = 64
DP = 128
L = 16
NC, NS = 2, 16
NW = NC * NS
SCALE = 8.0
NBUF = 4

B = 4096 * 200
BPW = B // NW
CHUNK = 128
NSTEP = BPW // CHUNK

_mesh = plsc.VectorSubcoreMesh(core_axis_name="c", subcore_axis_name="s")


@functools.partial(
    pl.kernel,
    out_type=jax.ShapeDtypeStruct((B, D), jnp.float32),
    mesh=_mesh,
    scratch_types=(
        [pltpu.VMEM((CHUNK,), jnp.int32) for _ in range(NBUF)]
        + [pltpu.VMEM((CHUNK, DP), jnp.float32) for _ in range(NBUF)]
        + [pltpu.VMEM((CHUNK, D), jnp.float32) for _ in range(2)]
        + [pltpu.SemaphoreType.DMA for _ in range(2 * NBUF + 2)]
    ),
    compiler_params=pltpu.CompilerParams(use_tc_tiling_on_sc=True),
)
def _emb_lookup(idx_hbm, table_hbm, out_hbm, *bufs):
    idxb = bufs[0:NBUF]
    rowsb = bufs[NBUF:2 * NBUF]
    outb = bufs[2 * NBUF:2 * NBUF + 2]
    isem = bufs[2 * NBUF + 2:3 * NBUF + 2]
    gsem = bufs[3 * NBUF + 2:4 * NBUF + 2]
    osem = bufs[4 * NBUF + 2:4 * NBUF + 4]

    wid = lax.axis_index("s") * NC + lax.axis_index("c")
    base = wid * BPW

    def idx_start(g, j):
        pltpu.async_copy(idx_hbm.at[pl.ds(base + g * CHUNK, CHUNK)],
                         idxb[j], isem[j])

    def idx_wait(j):
        pltpu.make_async_copy(idx_hbm.at[pl.ds(base, CHUNK)],
                              idxb[j], isem[j]).wait()

    def gather_start(j):
        pltpu.async_copy(table_hbm.at[idxb[j]], rowsb[j], gsem[j])

    def gather_wait(j):
        pltpu.make_async_copy(table_hbm.at[idxb[j]], rowsb[j], gsem[j]).wait()

    def out_start(g, jo):
        pltpu.async_copy(outb[jo],
                         out_hbm.at[pl.ds(base + g * CHUNK, CHUNK)], osem[jo])

    def out_wait(jo):
        pltpu.make_async_copy(outb[jo],
                              out_hbm.at[pl.ds(base, CHUNK)], osem[jo]).wait()

    for j in range(NBUF):
        idx_start(j, j)
    idx_wait(0)
    gather_start(0)
    idx_wait(1)
    gather_start(1)

    def chunk_body(g, j, jo):
        gather_wait(j)

        @pl.when(g + NBUF < NSTEP)
        def _():
            idx_start(g + NBUF, j)

        @pl.when(g + 2 < NSTEP)
        def _():
            j2 = (j + 2) % NBUF
            idx_wait(j2)
            gather_start(j2)

        @pl.when(g >= 2)
        def _():
            out_wait(jo)

        @plsc.parallel_loop(0, CHUNK, 1, unroll=4)
        def _(r):
            for c in range(D // L):
                sl = pl.ds(c * L, L)
                outb[jo][r, sl] = rowsb[j][r, sl]

        out_start(g, jo)

    @pl.loop(0, NSTEP, step=NBUF)
    def _(gg):
        for j in range(NBUF):
            chunk_body(gg + j, j, j % 2)

    out_wait(0)
    out_wait(1)


VOCAB = 1000000
BLKV = 4096


def _tc_transpose_kernel(wt_ref, out_ref):
    t = jnp.transpose(wt_ref[...] * SCALE, (1, 0))
    out_ref[:, 0:D] = t


def _tc_format_table(wt):
    grid = pl.cdiv(VOCAB, BLKV)
    return pl.pallas_call(
        _tc_transpose_kernel,
        out_shape=jax.ShapeDtypeStruct((VOCAB, DP), jnp.float32),
        grid=(grid,),
        in_specs=[pl.BlockSpec((D, BLKV), lambda i: (0, i))],
        out_specs=pl.BlockSpec((BLKV, DP), lambda i: (i, 0)),
    )(wt)


def kernel(x, weight):
    wp = _tc_format_table(weight.T)
    out = _emb_lookup(x.reshape(B), wp)
    return out.reshape(4096, 200, D)


# speedup vs baseline: 1.8203x; 1.0951x over previous
import functools

import jax
import jax.numpy as jnp
from jax import lax
from jax.experimental import pallas as pl
from jax.experimental.pallas import tpu as pltpu
from jax.experimental.pallas import tpu_sc as plsc

D = 64
DP = 128
L = 16
NC, NS = 2, 16
NW = NC * NS
SCALE = 8.0
NBUF = 4

B = 4096 * 200
BPW = B // NW
CHUNK = 160
NSTEP = BPW // CHUNK

_mesh = plsc.VectorSubcoreMesh(core_axis_name="c", subcore_axis_name="s")


@functools.partial(
    pl.kernel,
    out_type=jax.ShapeDtypeStruct((B, D), jnp.float32),
    mesh=_mesh,
    scratch_types=(
        [pltpu.VMEM((CHUNK,), jnp.int32) for _ in range(NBUF)]
        + [pltpu.VMEM((CHUNK, DP), jnp.float32) for _ in range(NBUF)]
        + [pltpu.VMEM((CHUNK, D), jnp.float32) for _ in range(2)]
        + [pltpu.SemaphoreType.DMA for _ in range(2 * NBUF + 2)]
    ),
    compiler_params=pltpu.CompilerParams(use_tc_tiling_on_sc=True),
)
def _emb_lookup(idx_hbm, table_hbm, out_hbm, *bufs):
    idxb = bufs[0:NBUF]
    rowsb = bufs[NBUF:2 * NBUF]
    outb = bufs[2 * NBUF:2 * NBUF + 2]
    isem = bufs[2 * NBUF + 2:3 * NBUF + 2]
    gsem = bufs[3 * NBUF + 2:4 * NBUF + 2]
    osem = bufs[4 * NBUF + 2:4 * NBUF + 4]

    wid = lax.axis_index("s") * NC + lax.axis_index("c")
    base = wid * BPW

    def idx_start(g, j):
        pltpu.async_copy(idx_hbm.at[pl.ds(base + g * CHUNK, CHUNK)],
                         idxb[j], isem[j])

    def idx_wait(j):
        pltpu.make_async_copy(idx_hbm.at[pl.ds(base, CHUNK)],
                              idxb[j], isem[j]).wait()

    def gather_start(j):
        pltpu.async_copy(table_hbm.at[idxb[j]], rowsb[j], gsem[j])

    def gather_wait(j):
        pltpu.make_async_copy(table_hbm.at[idxb[j]], rowsb[j], gsem[j]).wait()

    def out_start(g, jo):
        pltpu.async_copy(outb[jo],
                         out_hbm.at[pl.ds(base + g * CHUNK, CHUNK)], osem[jo])

    def out_wait(jo):
        pltpu.make_async_copy(outb[jo],
                              out_hbm.at[pl.ds(base, CHUNK)], osem[jo]).wait()

    for j in range(NBUF):
        idx_start(j, j)
    idx_wait(0)
    gather_start(0)
    idx_wait(1)
    gather_start(1)

    def chunk_body(g, j, jo):
        gather_wait(j)

        @pl.when(g + NBUF < NSTEP)
        def _():
            idx_start(g + NBUF, j)

        @pl.when(g + 2 < NSTEP)
        def _():
            j2 = (j + 2) % NBUF
            idx_wait(j2)
            gather_start(j2)

        @pl.when(g >= 2)
        def _():
            out_wait(jo)

        @plsc.parallel_loop(0, CHUNK, 1, unroll=4)
        def _(r):
            for c in range(D // L):
                sl = pl.ds(c * L, L)
                outb[jo][r, sl] = rowsb[j][r, sl]

        out_start(g, jo)

    @pl.loop(0, NSTEP, step=NBUF)
    def _(gg):
        for j in range(NBUF):
            chunk_body(gg + j, j, j % 2)

    out_wait(0)
    out_wait(1)


VOCAB = 1000000
BLKV = 8192


def _tc_transpose_kernel(wt_ref, out_ref):
    out_ref[:, 0:D] = jnp.transpose(wt_ref[...] * SCALE, (1, 0))


def _tc_format_table(wt):
    grid = pl.cdiv(VOCAB, BLKV)
    return pl.pallas_call(
        _tc_transpose_kernel,
        out_shape=jax.ShapeDtypeStruct((VOCAB, DP), jnp.float32),
        grid=(grid,),
        in_specs=[pl.BlockSpec((D, BLKV), lambda i: (0, i))],
        out_specs=pl.BlockSpec((BLKV, DP), lambda i: (i, 0)),
    )(wt)


def kernel(x, weight):
    wp = _tc_format_table(weight.T)
    out = _emb_lookup(x.reshape(B), wp)
    return out.reshape(4096, 200, D)
